# baseline (speedup 1.0000x reference)
import jax
import jax.numpy as jnp
from jax import lax
from jax.experimental import pallas as pl
from jax.experimental.pallas import tpu as pltpu
from jax.experimental.pallas import tpu_sc as plsc

N = 10000
E = 320000
D = 128
H = 128
C = 64

NC = 2
NS = 16
NW = NC * NS
K = 128
NCHUNKS = 80
E_PAD = NCHUNKS * NW * K
ROWS_PER_TILE = 640
NACC = ROWS_PER_TILE * NS
DROWS = NACC // 16
DCHUNK = 128

_mesh = plsc.VectorSubcoreMesh(
    core_axis_name="c", subcore_axis_name="s", num_cores=NC, num_subcores=NS)


DW = 16


NBUF = 2
NBUFD = 4


def _agg_body(x_hbm, idx_hbm, agg_out, acc_sh, src_all, zbuf_v, *bufs):
    dstb = bufs[0:NBUF]
    rows = bufs[NBUF:2 * NBUF]
    semi = bufs[2 * NBUF:3 * NBUF]
    semg = bufs[3 * NBUF:4 * NBUF]
    cid = lax.axis_index("c")
    sid = lax.axis_index("s")
    wid = sid * NC + cid

    zeros16 = jnp.zeros((16,), jnp.float32)

    for r in range(16):
        for cc in range(8):
            zbuf_v[r, pl.ds(cc * 16, 16)] = zeros16

    base_r = sid * ROWS_PER_TILE

    def zacc(i, carry):
        pltpu.sync_copy(zbuf_v, acc_sh.at[pl.ds(base_r + i * 16, 16)])
        return carry
    lax.fori_loop(0, ROWS_PER_TILE // 16, zacc, 0)

    ebase = wid * (NCHUNKS * K)
    pltpu.sync_copy(idx_hbm.at[0, pl.ds(ebase, NCHUNKS * K)], src_all)
    plsc.subcore_barrier()

    def gather(c, b):
        return pltpu.async_copy(x_hbm.at[src_all.at[pl.ds(c * K, K)]],
                                rows[b], semg[b])

    def dstld(c, b):
        return pltpu.async_copy(idx_hbm.at[1, pl.ds(ebase + c * K, K)],
                                dstb[b], semi[b])

    dstld(0, 0)
    gather(0, 0)
    dstld(1, 1)

    def body(i, carry):
        a = 2 * i
        b = a + 1
        gather(b, 1)
        pltpu.make_async_copy(x_hbm.at[src_all.at[pl.ds(0, K)]],
                              rows[0], semg[0]).wait()
        pltpu.make_async_copy(idx_hbm.at[1, pl.ds(ebase, K)],
                              dstb[0], semi[0]).wait()
        pltpu.sync_copy(rows[0], acc_sh.at[dstb[0]], add=True)
        gather(a + 2, 0)
        dstld(a + 2, 0)
        pltpu.make_async_copy(x_hbm.at[src_all.at[pl.ds(0, K)]],
                              rows[1], semg[1]).wait()
        pltpu.make_async_copy(idx_hbm.at[1, pl.ds(ebase, K)],
                              dstb[1], semi[1]).wait()
        pltpu.sync_copy(rows[1], acc_sh.at[dstb[1]], add=True)
        dstld(b + 2, 1)
        return carry
    lax.fori_loop(0, NCHUNKS // 2 - 1, body, 0)

    a = NCHUNKS - 2
    b = NCHUNKS - 1
    gather(b, 1)
    pltpu.make_async_copy(x_hbm.at[src_all.at[pl.ds(0, K)]],
                          rows[0], semg[0]).wait()
    pltpu.make_async_copy(idx_hbm.at[1, pl.ds(ebase, K)],
                          dstb[0], semi[0]).wait()
    pltpu.sync_copy(rows[0], acc_sh.at[dstb[0]], add=True)
    pltpu.make_async_copy(x_hbm.at[src_all.at[pl.ds(0, K)]],
                          rows[1], semg[1]).wait()
    pltpu.make_async_copy(idx_hbm.at[1, pl.ds(ebase, K)],
                          dstb[1], semi[1]).wait()
    pltpu.sync_copy(rows[1], acc_sh.at[dstb[1]], add=True)
    plsc.subcore_barrier()

    pltpu.sync_copy(acc_sh.at[pl.ds(base_r, ROWS_PER_TILE)],
                    agg_out.at[cid, pl.ds(base_r, ROWS_PER_TILE)])


_agg = pl.kernel(
    _agg_body,
    out_type=jax.ShapeDtypeStruct((NC, NACC, D), jnp.float32),
    mesh=_mesh,
    scratch_types=[
        pltpu.VMEM_SHARED((NACC, D), jnp.float32),
        pltpu.VMEM((NCHUNKS * K,), jnp.int32),
        pltpu.VMEM((16, D), jnp.float32),
    ] + [pltpu.VMEM((K,), jnp.int32) for _ in range(NBUF)]
      + [pltpu.VMEM((K, D), jnp.float32) for _ in range(NBUF)]
      + [pltpu.SemaphoreType.DMA for _ in range(2 * NBUF)],
)


def _deg_body(idx_hbm, deg_out, deg_sh, ones_v, zbuf_v, *bufs):
    dstb = bufs[0:NBUFD]
    semi = bufs[NBUFD:2 * NBUFD]
    sems = bufs[2 * NBUFD:3 * NBUFD]
    cid = lax.axis_index("c")
    sid = lax.axis_index("s")
    wid = sid * NC + cid

    zeros16 = jnp.zeros((16,), jnp.float32)
    ones16 = jnp.ones((16,), jnp.float32)
    for r in range(16):
        for cc in range(8):
            zbuf_v[r, pl.ds(cc * 16, 16)] = zeros16

    def fill_ones(i, carry):
        for cc in range(8):
            ones_v[i, pl.ds(cc * 16, 16)] = ones16
        return carry
    lax.fori_loop(0, K, fill_ones, 0)

    base_r = sid * ROWS_PER_TILE

    def zd(i, carry):
        pltpu.sync_copy(zbuf_v, deg_sh.at[pl.ds(base_r + i * 16, 16)])
        return carry
    lax.fori_loop(0, ROWS_PER_TILE // 16, zd, 0)

    ebase = wid * (NCHUNKS * K)

    def dstld(c, b):
        return pltpu.async_copy(idx_hbm.at[1, pl.ds(ebase + c * K, K)],
                                dstb[b], semi[b])
    plsc.subcore_barrier()

    for b in range(NBUFD):
        dstld(b, b)

    def step(s, carry, prefetch):
        for b in range(NBUFD):
            pltpu.make_async_copy(idx_hbm.at[1, pl.ds(ebase, K)],
                                  dstb[b], semi[b]).wait()
            pltpu.async_copy(ones_v, deg_sh.at[dstb[b]], sems[b], add=True)
        for b in range(NBUFD):
            c = NBUFD * s + b
            pltpu.make_async_copy(ones_v, deg_sh.at[dstb[b]], sems[b]).wait()
            if prefetch:
                dstld(c + NBUFD, b)
        return carry

    lax.fori_loop(0, NCHUNKS // NBUFD - 1,
                  lambda i, cy: step(i, cy, True), 0)
    step(NCHUNKS // NBUFD - 1, 0, False)
    plsc.subcore_barrier()

    pltpu.sync_copy(deg_sh.at[pl.ds(base_r, ROWS_PER_TILE)],
                    deg_out.at[cid, pl.ds(base_r, ROWS_PER_TILE)])


_deg = pl.kernel(
    _deg_body,
    out_type=jax.ShapeDtypeStruct((NC, NACC, D), jnp.float32),
    mesh=_mesh,
    scratch_types=[
        pltpu.VMEM_SHARED((NACC, D), jnp.float32),
        pltpu.VMEM((K, D), jnp.float32),
        pltpu.VMEM((16, D), jnp.float32),
    ] + [pltpu.VMEM((K,), jnp.int32) for _ in range(NBUFD)]
      + [pltpu.SemaphoreType.DMA for _ in range(2 * NBUFD)],
)


BN = 1024
NB = -(-N // BN)


def _lin_body(x_ref, w_ref, b_ref, out_ref):
    h = lax.dot_general(x_ref[...], w_ref[...], (((1,), (1,)), ((), ())),
                        preferred_element_type=jnp.float32)
    out_ref[...] = h + b_ref[...]


def _sage1_body(agg_ref, deg_ref, xr_ref, wl_ref, out_ref):
    agg = agg_ref[0] + agg_ref[1]
    deg = deg_ref[0] + deg_ref[1]
    dinv = 1.0 / jnp.maximum(deg, 1.0)
    mean = agg * dinv[:, None]
    h = lax.dot_general(mean, wl_ref[...], (((1,), (1,)), ((), ())),
                        preferred_element_type=jnp.float32)
    out_ref[...] = jnp.maximum(h + xr_ref[...], 0.0)


def _sage2_body(agg_ref, deg_ref, xr_ref, wl_ref,
                m1_ref, bm1_ref, m2_ref, bm2_ref, out_ref):
    agg = agg_ref[0] + agg_ref[1]
    deg = deg_ref[0] + deg_ref[1]
    dinv = 1.0 / jnp.maximum(deg, 1.0)
    mean = agg * dinv[:, None]
    h = lax.dot_general(mean, wl_ref[...], (((1,), (1,)), ((), ())),
                        preferred_element_type=jnp.float32)
    h = jnp.maximum(h + xr_ref[...], 0.0)
    h = lax.dot_general(h, m1_ref[...], (((1,), (1,)), ((), ())),
                        preferred_element_type=jnp.float32)
    h = jnp.maximum(h + bm1_ref[...], 0.0)
    h = lax.dot_general(h, m2_ref[...], (((1,), (1,)), ((), ())),
                        preferred_element_type=jnp.float32)
    h = h + bm2_ref[...]
    out_ref[...] = 1.0 / (1.0 + jnp.exp(-h))


def _full(shape):
    nd = len(shape)
    return pl.BlockSpec(shape, lambda i: (0,) * nd)


_lin = pl.pallas_call(
    _lin_body,
    grid=(NB,),
    in_specs=[
        pl.BlockSpec((BN, D), lambda i: (i, 0)),
        _full((H, D)),
        _full((1, H)),
    ],
    out_specs=pl.BlockSpec((BN, H), lambda i: (i, 0)),
    out_shape=jax.ShapeDtypeStruct((N, H), jnp.float32),
)

_sage1 = pl.pallas_call(
    _sage1_body,
    grid=(NB,),
    in_specs=[
        pl.BlockSpec((NC, BN, D), lambda i: (0, i, 0)),
        pl.BlockSpec((NC, BN), lambda i: (0, i)),
        pl.BlockSpec((BN, H), lambda i: (i, 0)),
        _full((H, D)),
    ],
    out_specs=pl.BlockSpec((BN, H), lambda i: (i, 0)),
    out_shape=jax.ShapeDtypeStruct((N, H), jnp.float32),
)

_sage2 = pl.pallas_call(
    _sage2_body,
    grid=(NB,),
    in_specs=[
        pl.BlockSpec((NC, BN, H), lambda i: (0, i, 0)),
        pl.BlockSpec((NC, BN), lambda i: (0, i)),
        pl.BlockSpec((BN, H), lambda i: (i, 0)),
        _full((H, H)),
        _full((H, H)),
        _full((1, H)),
        _full((C, H)),
        _full((1, C)),
    ],
    out_specs=pl.BlockSpec((BN, C), lambda i: (i, 0)),
    out_shape=jax.ShapeDtypeStruct((N, C), jnp.float32),
)


@jax.jit
def kernel(x, edge_index, W1l, W1r, b1, W2l, W2r, b2, M1, bm1, M2, bm2):
    pad = E_PAD - E
    fill_src = jnp.arange(pad, dtype=jnp.int32) % N
    fill_dst = N + (jnp.arange(pad, dtype=jnp.int32) % (NACC - N))
    idx2 = jnp.concatenate(
        [edge_index, jnp.stack([fill_src, fill_dst])], axis=1)

    deg3 = _deg(idx2)
    agg1 = _agg(x, idx2)
    xr1 = _lin(x, W1r, b1.reshape(1, H))
    deg = deg3[:, :, 0]
    h1 = _sage1(agg1, deg, xr1, W1l)
    agg2 = _agg(h1, idx2)
    xr2 = _lin(h1, W2r, b2.reshape(1, H))
    out = _sage2(agg2, deg, xr2, W2l,
                 M1, bm1.reshape(1, H), M2, bm2.reshape(1, C))
    return out


# speedup vs baseline: 1.0108x; 1.0108x over previous
import jax
import jax.numpy as jnp
from jax import lax
from jax.experimental import pallas as pl
from jax.experimental.pallas import tpu as pltpu
from jax.experimental.pallas import tpu_sc as plsc

N = 10000
E = 320000
D = 128
H = 128
C = 64

NC = 2
NS = 16
NW = NC * NS
K = 128
NCHUNKS = 78
NXTRA = E // K - NW * NCHUNKS
SRCPRE = NCHUNKS + 1
ROWS_PER_TILE = 640
NACC = ROWS_PER_TILE * NS
DROWS = NACC // 16
DCHUNK = 128

_mesh = plsc.VectorSubcoreMesh(
    core_axis_name="c", subcore_axis_name="s", num_cores=NC, num_subcores=NS)


DW = 16


NBUF = 2
NBUFD = 4


def _agg_body(x_hbm, idx_hbm, agg_out, acc_sh, src_all, zbuf_v, *bufs):
    dstb = bufs[0:NBUF]
    rows = bufs[NBUF:2 * NBUF]
    semi = bufs[2 * NBUF:3 * NBUF]
    semg = bufs[3 * NBUF:4 * NBUF]
    cid = lax.axis_index("c")
    sid = lax.axis_index("s")
    wid = sid * NC + cid

    zeros16 = jnp.zeros((16,), jnp.float32)

    for r in range(16):
        for cc in range(8):
            zbuf_v[r, pl.ds(cc * 16, 16)] = zeros16

    base_r = sid * ROWS_PER_TILE

    def zacc(i, carry):
        pltpu.sync_copy(zbuf_v, acc_sh.at[pl.ds(base_r + i * 16, 16)])
        return carry
    lax.fori_loop(0, ROWS_PER_TILE // 16, zacc, 0)

    nch = NCHUNKS + jnp.where(wid < NXTRA, 1, 0)
    ebase = (wid * NCHUNKS + jnp.minimum(wid, NXTRA)) * K
    pltpu.sync_copy(idx_hbm.at[0, pl.ds(ebase, SRCPRE * K)], src_all)
    plsc.subcore_barrier()

    def gather(c, b):
        return pltpu.async_copy(x_hbm.at[src_all.at[pl.ds(c * K, K)]],
                                rows[b], semg[b])

    def dstld(c, b):
        return pltpu.async_copy(idx_hbm.at[1, pl.ds(ebase + c * K, K)],
                                dstb[b], semi[b])

    dstld(0, 0)
    gather(0, 0)
    dstld(1, 1)
    gather(1, 1)

    def half(a, b0, prefetch):
        pltpu.make_async_copy(x_hbm.at[src_all.at[pl.ds(0, K)]],
                              rows[b0], semg[b0]).wait()
        pltpu.make_async_copy(idx_hbm.at[1, pl.ds(ebase, K)],
                              dstb[b0], semi[b0]).wait()
        pltpu.sync_copy(rows[b0], acc_sh.at[dstb[b0]], add=True)
        if prefetch:
            gather(a + 2, b0)
            dstld(a + 2, b0)

    def body(i, carry):
        a = 2 * i
        half(a, 0, True)
        half(a + 1, 1, True)
        return carry
    lax.fori_loop(0, NCHUNKS // 2 - 1, body, 0)

    a = NCHUNKS - 2
    half(a, 0, False)
    half(a + 1, 1, False)

    @pl.when(wid < NXTRA)
    def _extra():
        c = NCHUNKS
        pltpu.sync_copy(idx_hbm.at[1, pl.ds(ebase + c * K, K)], dstb[1])
        pltpu.async_copy(x_hbm.at[src_all.at[pl.ds(c * K, K)]],
                         rows[0], semg[0]).wait()
        pltpu.sync_copy(rows[0], acc_sh.at[dstb[1]], add=True)
    plsc.subcore_barrier()

    pltpu.sync_copy(acc_sh.at[pl.ds(base_r, ROWS_PER_TILE)],
                    agg_out.at[cid, pl.ds(base_r, ROWS_PER_TILE)])


_agg = pl.kernel(
    _agg_body,
    out_type=jax.ShapeDtypeStruct((NC, NACC, D), jnp.float32),
    mesh=_mesh,
    scratch_types=[
        pltpu.VMEM_SHARED((NACC, D), jnp.float32),
        pltpu.VMEM((SRCPRE * K,), jnp.int32),
        pltpu.VMEM((16, D), jnp.float32),
    ] + [pltpu.VMEM((K,), jnp.int32) for _ in range(NBUF)]
      + [pltpu.VMEM((K, D), jnp.float32) for _ in range(NBUF)]
      + [pltpu.SemaphoreType.DMA for _ in range(2 * NBUF)],
)


def _deg_body(idx_hbm, deg_out, deg_sh, ones_v, zbuf_v, *bufs):
    dstb = bufs[0:NBUFD]
    semi = bufs[NBUFD:2 * NBUFD]
    sems = bufs[2 * NBUFD:3 * NBUFD]
    cid = lax.axis_index("c")
    sid = lax.axis_index("s")
    wid = sid * NC + cid

    zeros16 = jnp.zeros((16,), jnp.float32)
    ones16 = jnp.ones((16,), jnp.float32)
    for r in range(16):
        for cc in range(8):
            zbuf_v[r, pl.ds(cc * 16, 16)] = zeros16

    def fill_ones(i, carry):
        for cc in range(8):
            ones_v[i, pl.ds(cc * 16, 16)] = ones16
        return carry
    lax.fori_loop(0, K, fill_ones, 0)

    base_r = sid * ROWS_PER_TILE

    def zd(i, carry):
        pltpu.sync_copy(zbuf_v, deg_sh.at[pl.ds(base_r + i * 16, 16)])
        return carry
    lax.fori_loop(0, ROWS_PER_TILE // 16, zd, 0)

    nch = NCHUNKS + jnp.where(wid < NXTRA, 1, 0)
    ebase = (wid * NCHUNKS + jnp.minimum(wid, NXTRA)) * K

    def dstld(c, b):
        return pltpu.async_copy(idx_hbm.at[1, pl.ds(ebase + c * K, K)],
                                dstb[b], semi[b])
    plsc.subcore_barrier()

    for b in range(NBUFD):
        dstld(b, b)

    def step(s, carry, prefetch):
        for b in range(NBUFD):
            pltpu.make_async_copy(idx_hbm.at[1, pl.ds(ebase, K)],
                                  dstb[b], semi[b]).wait()
            pltpu.async_copy(ones_v, deg_sh.at[dstb[b]], sems[b], add=True)
        for b in range(NBUFD):
            c = NBUFD * s + b
            pltpu.make_async_copy(ones_v, deg_sh.at[dstb[b]], sems[b]).wait()
            if prefetch:
                dstld(c + NBUFD, b)
        return carry

    lax.fori_loop(0, NCHUNKS // NBUFD - 1,
                  lambda i, cy: step(i, cy, True), 0)
    step(NCHUNKS // NBUFD - 1, 0, False)
    for b in range(NCHUNKS - NBUFD * (NCHUNKS // NBUFD)):
        c = NBUFD * (NCHUNKS // NBUFD) + b
        pltpu.sync_copy(idx_hbm.at[1, pl.ds(ebase + c * K, K)], dstb[b])
        pltpu.sync_copy(ones_v, deg_sh.at[dstb[b]], add=True)

    @pl.when(wid < NXTRA)
    def _extra():
        c = NCHUNKS
        pltpu.sync_copy(idx_hbm.at[1, pl.ds(ebase + c * K, K)], dstb[0])
        pltpu.sync_copy(ones_v, deg_sh.at[dstb[0]], add=True)
    plsc.subcore_barrier()

    pltpu.sync_copy(deg_sh.at[pl.ds(base_r, ROWS_PER_TILE)],
                    deg_out.at[cid, pl.ds(base_r, ROWS_PER_TILE)])


_deg = pl.kernel(
    _deg_body,
    out_type=jax.ShapeDtypeStruct((NC, NACC, D), jnp.float32),
    mesh=_mesh,
    scratch_types=[
        pltpu.VMEM_SHARED((NACC, D), jnp.float32),
        pltpu.VMEM((K, D), jnp.float32),
        pltpu.VMEM((16, D), jnp.float32),
    ] + [pltpu.VMEM((K,), jnp.int32) for _ in range(NBUFD)]
      + [pltpu.SemaphoreType.DMA for _ in range(2 * NBUFD)],
)


BN = 1024
NB = -(-N // BN)


def _lin_body(x_ref, w_ref, b_ref, out_ref):
    h = lax.dot_general(x_ref[...], w_ref[...], (((1,), (1,)), ((), ())),
                        preferred_element_type=jnp.float32)
    out_ref[...] = h + b_ref[...]


def _sage1_body(agg_ref, deg_ref, xr_ref, wl_ref, out_ref):
    agg = agg_ref[0] + agg_ref[1]
    deg = deg_ref[0] + deg_ref[1]
    dinv = 1.0 / jnp.maximum(deg, 1.0)
    mean = agg * dinv[:, None]
    h = lax.dot_general(mean, wl_ref[...], (((1,), (1,)), ((), ())),
                        preferred_element_type=jnp.float32)
    out_ref[...] = jnp.maximum(h + xr_ref[...], 0.0)


def _sage2_body(agg_ref, deg_ref, xr_ref, wl_ref,
                m1_ref, bm1_ref, m2_ref, bm2_ref, out_ref):
    agg = agg_ref[0] + agg_ref[1]
    deg = deg_ref[0] + deg_ref[1]
    dinv = 1.0 / jnp.maximum(deg, 1.0)
    mean = agg * dinv[:, None]
    h = lax.dot_general(mean, wl_ref[...], (((1,), (1,)), ((), ())),
                        preferred_element_type=jnp.float32)
    h = jnp.maximum(h + xr_ref[...], 0.0)
    h = lax.dot_general(h, m1_ref[...], (((1,), (1,)), ((), ())),
                        preferred_element_type=jnp.float32)
    h = jnp.maximum(h + bm1_ref[...], 0.0)
    h = lax.dot_general(h, m2_ref[...], (((1,), (1,)), ((), ())),
                        preferred_element_type=jnp.float32)
    h = h + bm2_ref[...]
    out_ref[...] = 1.0 / (1.0 + jnp.exp(-h))


def _full(shape):
    nd = len(shape)
    return pl.BlockSpec(shape, lambda i: (0,) * nd)


_lin = pl.pallas_call(
    _lin_body,
    grid=(NB,),
    in_specs=[
        pl.BlockSpec((BN, D), lambda i: (i, 0)),
        _full((H, D)),
        _full((1, H)),
    ],
    out_specs=pl.BlockSpec((BN, H), lambda i: (i, 0)),
    out_shape=jax.ShapeDtypeStruct((N, H), jnp.float32),
)

_sage1 = pl.pallas_call(
    _sage1_body,
    grid=(NB,),
    in_specs=[
        pl.BlockSpec((NC, BN, D), lambda i: (0, i, 0)),
        pl.BlockSpec((NC, BN), lambda i: (0, i)),
        pl.BlockSpec((BN, H), lambda i: (i, 0)),
        _full((H, D)),
    ],
    out_specs=pl.BlockSpec((BN, H), lambda i: (i, 0)),
    out_shape=jax.ShapeDtypeStruct((N, H), jnp.float32),
)

_sage2 = pl.pallas_call(
    _sage2_body,
    grid=(NB,),
    in_specs=[
        pl.BlockSpec((NC, BN, H), lambda i: (0, i, 0)),
        pl.BlockSpec((NC, BN), lambda i: (0, i)),
        pl.BlockSpec((BN, H), lambda i: (i, 0)),
        _full((H, H)),
        _full((H, H)),
        _full((1, H)),
        _full((C, H)),
        _full((1, C)),
    ],
    out_specs=pl.BlockSpec((BN, C), lambda i: (i, 0)),
    out_shape=jax.ShapeDtypeStruct((N, C), jnp.float32),
)


@jax.jit
def kernel(x, edge_index, W1l, W1r, b1, W2l, W2r, b2, M1, bm1, M2, bm2):
    deg3 = _deg(edge_index)
    agg1 = _agg(x, edge_index)
    xr1 = _lin(x, W1r, b1.reshape(1, H))
    deg = deg3[:, :, 0]
    h1 = _sage1(agg1, deg, xr1, W1l)
    agg2 = _agg(h1, edge_index)
    xr2 = _lin(h1, W2r, b2.reshape(1, H))
    out = _sage2(agg2, deg, xr2, W2l,
                 M1, bm1.reshape(1, H), M2, bm2.reshape(1, C))
    return out


# speedup vs baseline: 1.0124x; 1.0016x over previous
import jax
import jax.numpy as jnp
from jax import lax
from jax.experimental import pallas as pl
from jax.experimental.pallas import tpu as pltpu
from jax.experimental.pallas import tpu_sc as plsc

N = 10000
E = 320000
D = 128
H = 128
C = 64

NC = 2
NS = 16
NW = NC * NS
K = 128
NCHUNKS = 78
NXTRA = E // K - NW * NCHUNKS
SRCPRE = NCHUNKS + 1
ROWS_PER_TILE = 640
NACC = ROWS_PER_TILE * NS
_mesh = plsc.VectorSubcoreMesh(
    core_axis_name="c", subcore_axis_name="s", num_cores=NC, num_subcores=NS)


NBUF = 2
NBUFD = 4


def _agg_body(x_hbm, idx_hbm, agg_out, acc_sh, src_all, zbuf_v, *bufs):
    dstb = bufs[0:NBUF]
    rows = bufs[NBUF:2 * NBUF]
    semi = bufs[2 * NBUF:3 * NBUF]
    semg = bufs[3 * NBUF:4 * NBUF]
    cid = lax.axis_index("c")
    sid = lax.axis_index("s")
    wid = sid * NC + cid

    zeros16 = jnp.zeros((16,), jnp.float32)

    for r in range(16):
        for cc in range(8):
            zbuf_v[r, pl.ds(cc * 16, 16)] = zeros16

    base_r = sid * ROWS_PER_TILE

    def zacc(i, carry):
        pltpu.sync_copy(zbuf_v, acc_sh.at[pl.ds(base_r + i * 16, 16)])
        return carry
    lax.fori_loop(0, ROWS_PER_TILE // 16, zacc, 0)

    nch = NCHUNKS + jnp.where(wid < NXTRA, 1, 0)
    ebase = (wid * NCHUNKS + jnp.minimum(wid, NXTRA)) * K
    pltpu.sync_copy(idx_hbm.at[0, pl.ds(ebase, SRCPRE * K)], src_all)
    plsc.subcore_barrier()

    def gather(c, b):
        return pltpu.async_copy(x_hbm.at[src_all.at[pl.ds(c * K, K)]],
                                rows[b], semg[b])

    def dstld(c, b):
        return pltpu.async_copy(idx_hbm.at[1, pl.ds(ebase + c * K, K)],
                                dstb[b], semi[b])

    dstld(0, 0)
    gather(0, 0)
    dstld(1, 1)
    gather(1, 1)

    def half(a, b0, prefetch):
        pltpu.make_async_copy(x_hbm.at[src_all.at[pl.ds(0, K)]],
                              rows[b0], semg[b0]).wait()
        pltpu.make_async_copy(idx_hbm.at[1, pl.ds(ebase, K)],
                              dstb[b0], semi[b0]).wait()
        pltpu.sync_copy(rows[b0], acc_sh.at[dstb[b0]], add=True)
        if prefetch:
            gather(a + 2, b0)
            dstld(a + 2, b0)

    def body(i, carry):
        a = 2 * i
        half(a, 0, True)
        half(a + 1, 1, True)
        return carry
    lax.fori_loop(0, NCHUNKS // 2 - 1, body, 0)

    a = NCHUNKS - 2
    half(a, 0, False)
    half(a + 1, 1, False)

    @pl.when(wid < NXTRA)
    def _extra():
        c = NCHUNKS
        pltpu.sync_copy(idx_hbm.at[1, pl.ds(ebase + c * K, K)], dstb[1])
        pltpu.async_copy(x_hbm.at[src_all.at[pl.ds(c * K, K)]],
                         rows[0], semg[0]).wait()
        pltpu.sync_copy(rows[0], acc_sh.at[dstb[1]], add=True)
    plsc.subcore_barrier()

    pltpu.sync_copy(acc_sh.at[pl.ds(base_r, ROWS_PER_TILE)],
                    agg_out.at[cid, pl.ds(base_r, ROWS_PER_TILE)])


_agg = pl.kernel(
    _agg_body,
    out_type=jax.ShapeDtypeStruct((NC, NACC, D), jnp.float32),
    mesh=_mesh,
    scratch_types=[
        pltpu.VMEM_SHARED((NACC, D), jnp.float32),
        pltpu.VMEM((SRCPRE * K,), jnp.int32),
        pltpu.VMEM((16, D), jnp.float32),
    ] + [pltpu.VMEM((K,), jnp.int32) for _ in range(NBUF)]
      + [pltpu.VMEM((K, D), jnp.float32) for _ in range(NBUF)]
      + [pltpu.SemaphoreType.DMA for _ in range(2 * NBUF)],
)


def _deg_body(idx_hbm, deg_out, deg_sh, ones_v, zbuf_v, *bufs):
    dstb = bufs[0:NBUFD]
    semi = bufs[NBUFD:2 * NBUFD]
    sems = bufs[2 * NBUFD:3 * NBUFD]
    cid = lax.axis_index("c")
    sid = lax.axis_index("s")
    wid = sid * NC + cid

    zeros16 = jnp.zeros((16,), jnp.float32)
    ones16 = jnp.ones((16,), jnp.float32)
    for r in range(16):
        for cc in range(8):
            zbuf_v[r, pl.ds(cc * 16, 16)] = zeros16

    def fill_ones(i, carry):
        for cc in range(8):
            ones_v[i, pl.ds(cc * 16, 16)] = ones16
        return carry
    lax.fori_loop(0, K, fill_ones, 0)

    base_r = sid * ROWS_PER_TILE

    def zd(i, carry):
        pltpu.sync_copy(zbuf_v, deg_sh.at[pl.ds(base_r + i * 16, 16)])
        return carry
    lax.fori_loop(0, ROWS_PER_TILE // 16, zd, 0)

    nch = NCHUNKS + jnp.where(wid < NXTRA, 1, 0)
    ebase = (wid * NCHUNKS + jnp.minimum(wid, NXTRA)) * K

    def dstld(c, b):
        return pltpu.async_copy(idx_hbm.at[1, pl.ds(ebase + c * K, K)],
                                dstb[b], semi[b])
    plsc.subcore_barrier()

    for b in range(NBUFD):
        dstld(b, b)

    def step(s, carry, prefetch):
        for b in range(NBUFD):
            pltpu.make_async_copy(idx_hbm.at[1, pl.ds(ebase, K)],
                                  dstb[b], semi[b]).wait()
            pltpu.async_copy(ones_v, deg_sh.at[dstb[b]], sems[b], add=True)
        for b in range(NBUFD):
            c = NBUFD * s + b
            pltpu.make_async_copy(ones_v, deg_sh.at[dstb[b]], sems[b]).wait()
            if prefetch:
                dstld(c + NBUFD, b)
        return carry

    lax.fori_loop(0, NCHUNKS // NBUFD - 1,
                  lambda i, cy: step(i, cy, True), 0)
    step(NCHUNKS // NBUFD - 1, 0, False)
    for b in range(NCHUNKS - NBUFD * (NCHUNKS // NBUFD)):
        c = NBUFD * (NCHUNKS // NBUFD) + b
        pltpu.sync_copy(idx_hbm.at[1, pl.ds(ebase + c * K, K)], dstb[b])
        pltpu.sync_copy(ones_v, deg_sh.at[dstb[b]], add=True)

    @pl.when(wid < NXTRA)
    def _extra():
        c = NCHUNKS
        pltpu.sync_copy(idx_hbm.at[1, pl.ds(ebase + c * K, K)], dstb[0])
        pltpu.sync_copy(ones_v, deg_sh.at[dstb[0]], add=True)
    plsc.subcore_barrier()

    pltpu.sync_copy(deg_sh.at[pl.ds(base_r, ROWS_PER_TILE)],
                    deg_out.at[cid, pl.ds(base_r, ROWS_PER_TILE)])


_deg = pl.kernel(
    _deg_body,
    out_type=jax.ShapeDtypeStruct((NC, NACC, D), jnp.float32),
    mesh=_mesh,
    scratch_types=[
        pltpu.VMEM_SHARED((NACC, D), jnp.float32),
        pltpu.VMEM((K, D), jnp.float32),
        pltpu.VMEM((16, D), jnp.float32),
    ] + [pltpu.VMEM((K,), jnp.int32) for _ in range(NBUFD)]
      + [pltpu.SemaphoreType.DMA for _ in range(2 * NBUFD)],
)


BN = 1024
NB = -(-N // BN)


def _lin_body(x_ref, w_ref, b_ref, out_ref):
    h = lax.dot_general(x_ref[...], w_ref[...], (((1,), (1,)), ((), ())),
                        preferred_element_type=jnp.float32)
    out_ref[...] = h + b_ref[...]


def _sage1_body(agg_ref, deg_ref, xr_ref, wl_ref, out_ref):
    agg = agg_ref[0] + agg_ref[1]
    deg = deg_ref[0] + deg_ref[1]
    dinv = 1.0 / jnp.maximum(deg, 1.0)
    mean = agg * dinv[:, None]
    h = lax.dot_general(mean, wl_ref[...], (((1,), (1,)), ((), ())),
                        preferred_element_type=jnp.float32)
    out_ref[...] = jnp.maximum(h + xr_ref[...], 0.0)


def _sage2_body(agg_ref, deg_ref, xr_ref, wl_ref,
                m1_ref, bm1_ref, m2_ref, bm2_ref, out_ref):
    agg = agg_ref[0] + agg_ref[1]
    deg = deg_ref[0] + deg_ref[1]
    dinv = 1.0 / jnp.maximum(deg, 1.0)
    mean = agg * dinv[:, None]
    h = lax.dot_general(mean, wl_ref[...], (((1,), (1,)), ((), ())),
                        preferred_element_type=jnp.float32)
    h = jnp.maximum(h + xr_ref[...], 0.0)
    h = lax.dot_general(h, m1_ref[...], (((1,), (1,)), ((), ())),
                        preferred_element_type=jnp.float32)
    h = jnp.maximum(h + bm1_ref[...], 0.0)
    h = lax.dot_general(h, m2_ref[...], (((1,), (1,)), ((), ())),
                        preferred_element_type=jnp.float32)
    h = h + bm2_ref[...]
    out_ref[...] = 1.0 / (1.0 + jnp.exp(-h))


def _full(shape):
    nd = len(shape)
    return pl.BlockSpec(shape, lambda i: (0,) * nd)


_lin = pl.pallas_call(
    _lin_body,
    grid=(NB,),
    in_specs=[
        pl.BlockSpec((BN, D), lambda i: (i, 0)),
        _full((H, D)),
        _full((1, H)),
    ],
    out_specs=pl.BlockSpec((BN, H), lambda i: (i, 0)),
    out_shape=jax.ShapeDtypeStruct((N, H), jnp.float32),
)

_sage1 = pl.pallas_call(
    _sage1_body,
    grid=(NB,),
    in_specs=[
        pl.BlockSpec((NC, BN, D), lambda i: (0, i, 0)),
        pl.BlockSpec((NC, BN), lambda i: (0, i)),
        pl.BlockSpec((BN, H), lambda i: (i, 0)),
        _full((H, D)),
    ],
    out_specs=pl.BlockSpec((BN, H), lambda i: (i, 0)),
    out_shape=jax.ShapeDtypeStruct((N, H), jnp.float32),
)

_sage2 = pl.pallas_call(
    _sage2_body,
    grid=(NB,),
    in_specs=[
        pl.BlockSpec((NC, BN, H), lambda i: (0, i, 0)),
        pl.BlockSpec((NC, BN), lambda i: (0, i)),
        pl.BlockSpec((BN, H), lambda i: (i, 0)),
        _full((H, H)),
        _full((H, H)),
        _full((1, H)),
        _full((C, H)),
        _full((1, C)),
    ],
    out_specs=pl.BlockSpec((BN, C), lambda i: (i, 0)),
    out_shape=jax.ShapeDtypeStruct((N, C), jnp.float32),
)


@jax.jit
def kernel(x, edge_index, W1l, W1r, b1, W2l, W2r, b2, M1, bm1, M2, bm2):
    deg3 = _deg(edge_index)
    agg1 = _agg(x, edge_index)
    xr1 = _lin(x, W1r, b1.reshape(1, H))
    deg = deg3[:, :, 0]
    h1 = _sage1(agg1, deg, xr1, W1l)
    agg2 = _agg(h1, edge_index)
    xr2 = _lin(h1, W2r, b2.reshape(1, H))
    out = _sage2(agg2, deg, xr2, W2l,
                 M1, bm1.reshape(1, H), M2, bm2.reshape(1, C))
    return out
